# Initial kernel scaffold; baseline (speedup 1.0000x reference)
import jax
import jax.numpy as jnp
from jax.experimental import pallas as pl


def kernel(x, W, b, Wg, lora_A, lora_B):
    raise NotImplementedError("write your pallas kernel here")


# speedup vs baseline: 18.3417x; 18.3417x over previous
import jax
import jax.numpy as jnp
from jax.experimental import pallas as pl

T = 4096
IN = 768
OUT = 768
E = 64
R = 8
ALPHA = 16.0
SCALING = ALPHA / R

TM = 512


def _fused_body(x_ref, W_ref, b_ref, Wg_ref, A_ref, B_ref, o_ref):
    x = x_ref[...]
    logits = jnp.dot(x, Wg_ref[...], preferred_element_type=jnp.float32)
    eidx = jax.lax.broadcasted_iota(jnp.int32, (TM, E), 1)
    m1 = jnp.max(logits, axis=1, keepdims=True)
    a1 = jnp.min(jnp.where(logits == m1, eidx, E), axis=1, keepdims=True)
    masked = jnp.where(eidx == a1, -1e30, logits)
    m2 = jnp.max(masked, axis=1, keepdims=True)
    a2 = jnp.min(jnp.where(masked == m2, eidx, E), axis=1, keepdims=True)
    e2 = jnp.exp(m2 - m1)
    g1 = 1.0 / (1.0 + e2)
    g2 = e2 / (1.0 + e2)
    ef = jax.lax.broadcasted_iota(jnp.int32, (TM, E * R), 1) // R
    w_full = jnp.where(ef == a1, g1, 0.0) + jnp.where(ef == a2, g2, 0.0)
    base = jnp.dot(x, W_ref[...], preferred_element_type=jnp.float32)
    H = jnp.dot(x, A_ref[...], preferred_element_type=jnp.float32)
    lo = jnp.dot(H * w_full, B_ref[...], preferred_element_type=jnp.float32)
    o_ref[...] = base + b_ref[...] + SCALING * lo


def kernel(x, W, b, Wg, lora_A, lora_B):
    A2d = lora_A.transpose(1, 0, 2).reshape(IN, E * R)
    B2d = lora_B.reshape(E * R, OUT)
    b2 = b.reshape(1, OUT)
    grid = (T // TM,)
    return pl.pallas_call(
        _fused_body,
        grid=grid,
        in_specs=[
            pl.BlockSpec((TM, IN), lambda i: (i, 0)),
            pl.BlockSpec((IN, OUT), lambda i: (0, 0)),
            pl.BlockSpec((1, OUT), lambda i: (0, 0)),
            pl.BlockSpec((IN, E), lambda i: (0, 0)),
            pl.BlockSpec((IN, E * R), lambda i: (0, 0)),
            pl.BlockSpec((E * R, OUT), lambda i: (0, 0)),
        ],
        out_specs=pl.BlockSpec((TM, OUT), lambda i: (i, 0)),
        out_shape=jax.ShapeDtypeStruct((T, OUT), jnp.float32),
    )(x, W, b2, Wg, A2d, B2d)


# speedup vs baseline: 18.5682x; 1.0123x over previous
import jax
import jax.numpy as jnp
from jax.experimental import pallas as pl

T = 4096
IN = 768
OUT = 768
E = 64
R = 8
ALPHA = 16.0
SCALING = ALPHA / R

TM = 512


def _fused_body(x_ref, W_ref, b_ref, Wg_ref, A_ref, B_ref, o_ref):
    x = x_ref[...]
    logits = jnp.dot(x, Wg_ref[...], preferred_element_type=jnp.float32)
    eidx = jax.lax.broadcasted_iota(jnp.int32, (TM, E), 1)
    m1 = jnp.max(logits, axis=1, keepdims=True)
    a1 = jnp.min(jnp.where(logits == m1, eidx, E), axis=1, keepdims=True)
    masked = jnp.where(eidx == a1, -1e30, logits)
    m2 = jnp.max(masked, axis=1, keepdims=True)
    a2 = jnp.min(jnp.where(masked == m2, eidx, E), axis=1, keepdims=True)
    e2 = jnp.exp(m2 - m1)
    g1 = 1.0 / (1.0 + e2)
    g2 = e2 / (1.0 + e2)
    ef = jax.lax.broadcasted_iota(jnp.int32, (TM, E * R), 1) // R
    w_full = jnp.where(ef == a1, g1, 0.0) + jnp.where(ef == a2, g2, 0.0)
    base = jnp.dot(x, W_ref[...], preferred_element_type=jnp.float32)
    xb = x.astype(jnp.bfloat16)
    H = jnp.dot(xb, A_ref[...].astype(jnp.bfloat16),
                preferred_element_type=jnp.float32)
    lo = jnp.dot((H * w_full).astype(jnp.bfloat16),
                 B_ref[...].astype(jnp.bfloat16),
                 preferred_element_type=jnp.float32)
    o_ref[...] = base + b_ref[...] + SCALING * lo


def kernel(x, W, b, Wg, lora_A, lora_B):
    A2d = lora_A.transpose(1, 0, 2).reshape(IN, E * R)
    B2d = lora_B.reshape(E * R, OUT)
    b2 = b.reshape(1, OUT)
    grid = (T // TM,)
    return pl.pallas_call(
        _fused_body,
        grid=grid,
        in_specs=[
            pl.BlockSpec((TM, IN), lambda i: (i, 0)),
            pl.BlockSpec((IN, OUT), lambda i: (0, 0)),
            pl.BlockSpec((1, OUT), lambda i: (0, 0)),
            pl.BlockSpec((IN, E), lambda i: (0, 0)),
            pl.BlockSpec((IN, E * R), lambda i: (0, 0)),
            pl.BlockSpec((E * R, OUT), lambda i: (0, 0)),
        ],
        out_specs=pl.BlockSpec((TM, OUT), lambda i: (i, 0)),
        out_shape=jax.ShapeDtypeStruct((T, OUT), jnp.float32),
    )(x, W, b2, Wg, A2d, B2d)
